# baseline (speedup 1.0000x reference)
import jax
import jax.numpy as jnp
from jax import lax
from jax.experimental import pallas as pl
from jax.experimental.pallas import tpu as pltpu
from jax.experimental.pallas import tpu_sc as plsc

B = 16384
NCAND = 20
D = 64
LANES = 16
NC = 2
NS = 16
NW = NC * NS
BPW = B // NW
CPW = BPW * NCAND
NROWS = 1000000
NPAIR = NROWS // 2
W = 2 * D

SLAB = 256
NFULL = NROWS // SLAB
TAIL_COL = NFULL * SLAB
TAIL_W = NROWS - TAIL_COL
QN = -(-NFULL // NW)
SGROUPS = SLAB // LANES

CHUNK = 128
NCHUNK = CPW // CHUNK
UCHUNK = BPW // CHUNK
GROUPS = CHUNK // LANES
NBUF = 2


def _tr_body(utT, itT, tail_u, tail_i, uP, iP, slab_v, sh_v, tail_v, *sems):
    cid = lax.axis_index("c")
    sid = lax.axis_index("s")
    wid = sid * NC + cid
    lane = lax.iota(jnp.int32, LANES)
    half = (lane % 2) * D
    tabs = ((utT, uP), (itT, iP))
    sin = sems[0:4]
    sout = sems[4:6]
    HP = SLAB // 2

    for u in range(4):
        tab = tabs[u % 2][0]
        s0 = wid + (u // 2) * NW
        pltpu.async_copy(tab.at[:, pl.ds(s0 * SLAB, SLAB)],
                         slab_v.at[u], sin[u])

    def ring(mm, carry):
        for u in range(4):
            tab, outP = tabs[u % 2]
            q = 2 * mm + (u // 2)
            s = wid + q * NW

            @pl.when(s < NFULL)
            def _process():
                pltpu.make_async_copy(tab.at[:, pl.ds(s * SLAB, SLAB)],
                                      slab_v.at[u], sin[u]).wait()
                @pl.when(s >= wid + NW)
                def _drain_prev():
                    pltpu.make_async_copy(
                        sh_v.at[u % 2],
                        outP.at[pl.ds((s - NW) * HP, HP)],
                        sout[u % 2]).wait()

                def group(g, c2):
                    c0 = g * LANES
                    pvec = (c0 + lane) // 2
                    for d in range(D):
                        ld = slab_v[u, d, pl.ds(c0, 16)]
                        plsc.store_scatter(sh_v.at[u % 2], [pvec, half + d],
                                           ld)
                    return c2

                lax.fori_loop(0, SGROUPS, group, 0)
                @pl.when(s + 2 * NW < NFULL)
                def _prefetch():
                    pltpu.async_copy(
                        tab.at[:, pl.ds((s + 2 * NW) * SLAB, SLAB)],
                        slab_v.at[u], sin[u])
                pltpu.async_copy(sh_v.at[u % 2],
                                 outP.at[pl.ds(s * HP, HP)], sout[u % 2])
        return carry

    lax.fori_loop(0, (QN + 1) // 2, ring, 0)

    jl = (NFULL - 1 - wid) // NW
    sl = wid + jl * NW
    for b, (tab, outP) in enumerate(tabs):
        pltpu.make_async_copy(sh_v.at[b],
                              outP.at[pl.ds(sl * HP, HP)],
                              sout[b]).wait()

    @pl.when(wid == 1)
    def _tail():
        for tail_in, outP in ((tail_u, uP), (tail_i, iP)):
            pltpu.sync_copy(tail_in, tail_v)
            pltpu.sync_copy(tail_v,
                            outP.at[pl.ds(TAIL_COL // 2, TAIL_W // 2)])


def _dot_body(up_idx, uq_idx, ip_idx, iq_idx, uP, iP, out,
              uidx_v, uq_v, iidx_v, iq_v, urows_v, irows_v, scores_v,
              sem_u, *sems):
    cid = lax.axis_index("c")
    sid = lax.axis_index("s")
    wid = sid * NC + cid

    pltpu.sync_copy(up_idx.at[pl.ds(wid * UCHUNK, UCHUNK)], uidx_v)
    pltpu.sync_copy(uq_idx.at[pl.ds(wid * UCHUNK, UCHUNK)], uq_v)
    pltpu.sync_copy(ip_idx.at[pl.ds(wid * NCHUNK, NCHUNK)], iidx_v)
    pltpu.sync_copy(iq_idx.at[pl.ds(wid * NCHUNK, NCHUNK)], iq_v)

    for j in range(UCHUNK):
        buf = irows_v.at[j % NBUF]
        pltpu.async_copy(uP.at[uidx_v.at[j]], buf, sem_u).wait()

        def compact(r16, carry):
            qvec = uq_v[j, pl.ds(r16 * 16, 16)]
            for kk in range(16):
                r = r16 * 16 + kk
                qoff = qvec[kk]
                for k in range(D // 16):
                    urows_v[j * CHUNK + r, pl.ds(k * 16, 16)] = (
                        buf[r, pl.ds(qoff + k * 16, 16)])
            return carry

        lax.fori_loop(0, CHUNK // 16, compact, 0)

    for b in range(NBUF):
        pltpu.async_copy(iP.at[iidx_v.at[b]], irows_v.at[b], sems[b])

    lane = lax.iota(jnp.int32, LANES)

    def ring_body(jj, carry):
        for b in range(NBUF):
            j = jj * NBUF + b
            buf = irows_v.at[b]
            pltpu.make_async_copy(iP.at[iidx_v.at[j]], buf, sems[b]).wait()
            c_base = j * CHUNK
            for g in range(GROUPS):
                acc = jnp.zeros((LANES,), jnp.float32)
                qvec = iq_v[j, pl.ds(g * LANES, LANES)]
                for k in range(LANES):
                    r = g * LANES + k
                    bu = (c_base + r) // NCAND
                    qi = qvec[k]
                    t = (urows_v[bu, pl.ds(0, 16)] * buf[r, pl.ds(qi, 16)]
                         + urows_v[bu, pl.ds(16, 16)]
                         * buf[r, pl.ds(qi + 16, 16)]
                         + urows_v[bu, pl.ds(32, 16)]
                         * buf[r, pl.ds(qi + 32, 16)]
                         + urows_v[bu, pl.ds(48, 16)]
                         * buf[r, pl.ds(qi + 48, 16)])
                    acc = jnp.where(lane == k, jnp.sum(t), acc)
                scores_v[pl.ds(c_base + g * LANES, LANES)] = acc

            @pl.when(j + NBUF < NCHUNK)
            def _prefetch():
                pltpu.async_copy(iP.at[iidx_v.at[j + NBUF]], buf, sems[b])
        return carry

    lax.fori_loop(0, NCHUNK // NBUF, ring_body, 0)
    pltpu.sync_copy(scores_v, out.at[pl.ds(wid * CPW, CPW)])


def kernel(user, item, user_table, item_table):
    mesh = plsc.VectorSubcoreMesh(core_axis_name="c", subcore_axis_name="s")
    params = pltpu.CompilerParams(
        needs_layout_passes=False, use_tc_tiling_on_sc=True)

    uP, iP = pl.kernel(
        _tr_body,
        out_type=(jax.ShapeDtypeStruct((NPAIR, W), jnp.float32),
                  jax.ShapeDtypeStruct((NPAIR, W), jnp.float32)),
        mesh=mesh,
        compiler_params=params,
        scratch_types=[
            pltpu.VMEM((4, D, SLAB), jnp.float32),
            pltpu.VMEM((2, SLAB // 2, W), jnp.float32),
            pltpu.VMEM((TAIL_W // 2, W), jnp.float32),
            *([pltpu.SemaphoreType.DMA] * 6),
        ],
    )(user_table.T, item_table.T,
      user_table[TAIL_COL:].reshape(TAIL_W // 2, W),
      item_table[TAIL_COL:].reshape(TAIL_W // 2, W))

    item_f = item.reshape(-1)
    up2d = (user // 2).reshape(B // CHUNK, CHUNK)
    uq2d = ((user % 2) * D).reshape(B // CHUNK, CHUNK)
    ip2d = (item_f // 2).reshape((B * NCAND) // CHUNK, CHUNK)
    iq2d = ((item_f % 2) * D).reshape((B * NCAND) // CHUNK, CHUNK)

    scores = pl.kernel(
        _dot_body,
        out_type=jax.ShapeDtypeStruct((B * NCAND,), jnp.float32),
        mesh=mesh,
        compiler_params=params,
        scratch_types=[
            pltpu.VMEM((UCHUNK, CHUNK), jnp.int32),
            pltpu.VMEM((UCHUNK, CHUNK), jnp.int32),
            pltpu.VMEM((NCHUNK, CHUNK), jnp.int32),
            pltpu.VMEM((NCHUNK, CHUNK), jnp.int32),
            pltpu.VMEM((BPW, D), jnp.float32),
            pltpu.VMEM((NBUF, CHUNK, W), jnp.float32),
            pltpu.VMEM((CPW,), jnp.float32),
            pltpu.SemaphoreType.DMA,
            *([pltpu.SemaphoreType.DMA] * NBUF),
        ],
    )(up2d, uq2d, ip2d, iq2d, uP, iP)
    return scores.reshape(B, NCAND)


# speedup vs baseline: 1.7300x; 1.7300x over previous
import jax
import jax.numpy as jnp
from jax import lax
from jax.experimental import pallas as pl
from jax.experimental.pallas import tpu as pltpu
from jax.experimental.pallas import tpu_sc as plsc

B = 16384
NCAND = 20
D = 64
LANES = 16
NC = 2
NS = 16
NW = NC * NS
BPW = B // NW
CPW = BPW * NCAND
NROWS = 1000000
NPAIR = NROWS // 2
W = 2 * D

SLAB = 256
NFULL = NROWS // SLAB
TAIL_COL = NFULL * SLAB
TAIL_W = NROWS - TAIL_COL
QN = -(-NFULL // NW)
SGROUPS = SLAB // LANES
PITCH = 264
BUFW = D * PITCH

CHUNK = 128
NCHUNK = CPW // CHUNK
UCHUNK = BPW // CHUNK
GROUPS = CHUNK // LANES
NBUF = 2


def _tr_body(utT, itT, tail_u, tail_i, uP, iP, slab_v, sh_v, tail_v, *sems):
    cid = lax.axis_index("c")
    sid = lax.axis_index("s")
    wid = sid * NC + cid
    lane = lax.iota(jnp.int32, LANES)
    half = (lane % 2) * D
    tabs = ((utT, uP), (itT, iP))
    sin = sems[0:4]
    sout = sems[4:6]
    HP = SLAB // 2

    rvs = [(lane + r) % LANES for r in range(LANES)]

    def issue_slab(tab, u, s):
        pltpu.async_copy(tab.at[:, pl.ds(s * SLAB, SLAB)],
                         slab_v.at[u], sin[u])

    def drain_slab(tab, u, s):
        pltpu.make_async_copy(tab.at[:, pl.ds(s * SLAB, SLAB)],
                              slab_v.at[u], sin[u]).wait()

    for u in range(4):
        issue_slab(tabs[u % 2][0], u, wid + (u // 2) * NW)

    def ring(mm, carry):
        for u in range(4):
            tab, outP = tabs[u % 2]
            q = 2 * mm + (u // 2)
            s = wid + q * NW

            @pl.when(s < NFULL)
            def _process():
                drain_slab(tab, u, s)
                @pl.when(s >= wid + NW)
                def _drain_prev():
                    pltpu.make_async_copy(
                        sh_v.at[u % 2],
                        outP.at[pl.ds((s - NW) * HP, HP)],
                        sout[u % 2]).wait()

                sh = sh_v.at[u % 2]

                def group(g, c2):
                    u0 = g * LANES
                    uvec = u0 + lane
                    pvec = uvec // 2
                    for k in range(D // LANES):
                        for r in range(LANES):
                            dvec = rvs[r] + (k * LANES)
                            v = plsc.load_gather(slab_v.at[u], [dvec, uvec])
                            plsc.store_scatter(sh, [pvec, half + dvec], v)
                    return c2

                lax.fori_loop(0, SGROUPS, group, 0)
                @pl.when(s + 2 * NW < NFULL)
                def _prefetch():
                    issue_slab(tab, u, s + 2 * NW)
                pltpu.async_copy(sh_v.at[u % 2],
                                 outP.at[pl.ds(s * HP, HP)], sout[u % 2])
        return carry

    lax.fori_loop(0, (QN + 1) // 2, ring, 0)

    jl = (NFULL - 1 - wid) // NW
    sl = wid + jl * NW
    for b, (tab, outP) in enumerate(tabs):
        pltpu.make_async_copy(sh_v.at[b],
                              outP.at[pl.ds(sl * HP, HP)],
                              sout[b]).wait()

    @pl.when(wid == 1)
    def _tail():
        for tail_in, outP in ((tail_u, uP), (tail_i, iP)):
            pltpu.sync_copy(tail_in, tail_v)
            pltpu.sync_copy(tail_v,
                            outP.at[pl.ds(TAIL_COL // 2, TAIL_W // 2)])


def _dot_body(up_idx, uq_idx, ip_idx, iq_idx, uP, iP, out,
              uidx_v, uq_v, iidx_v, iq_v, urows_v, irows_v, scores_v,
              sem_u, *sems):
    cid = lax.axis_index("c")
    sid = lax.axis_index("s")
    wid = sid * NC + cid

    pltpu.sync_copy(up_idx.at[pl.ds(wid * UCHUNK, UCHUNK)], uidx_v)
    pltpu.sync_copy(uq_idx.at[pl.ds(wid * UCHUNK, UCHUNK)], uq_v)
    pltpu.sync_copy(ip_idx.at[pl.ds(wid * NCHUNK, NCHUNK)], iidx_v)
    pltpu.sync_copy(iq_idx.at[pl.ds(wid * NCHUNK, NCHUNK)], iq_v)

    for j in range(UCHUNK):
        buf = irows_v.at[j % NBUF]
        pltpu.async_copy(uP.at[uidx_v.at[j]], buf, sem_u).wait()

        def compact(r16, carry):
            qvec = uq_v[j, pl.ds(r16 * 16, 16)]
            for kk in range(16):
                r = r16 * 16 + kk
                qoff = qvec[kk]
                for k in range(D // 16):
                    urows_v[j * CHUNK + r, pl.ds(k * 16, 16)] = (
                        buf[r, pl.ds(qoff + k * 16, 16)])
            return carry

        lax.fori_loop(0, CHUNK // 16, compact, 0)

    for b in range(NBUF):
        pltpu.async_copy(iP.at[iidx_v.at[b]], irows_v.at[b], sems[b])

    lane = lax.iota(jnp.int32, LANES)

    def ring_body(jj, carry):
        for b in range(NBUF):
            j = jj * NBUF + b
            buf = irows_v.at[b]
            pltpu.make_async_copy(iP.at[iidx_v.at[j]], buf, sems[b]).wait()
            c_base = j * CHUNK
            for g in range(GROUPS):
                acc = jnp.zeros((LANES,), jnp.float32)
                qvec = iq_v[j, pl.ds(g * LANES, LANES)]
                for k in range(LANES):
                    r = g * LANES + k
                    bu = (c_base + r) // NCAND
                    qi = qvec[k]
                    t = (urows_v[bu, pl.ds(0, 16)] * buf[r, pl.ds(qi, 16)]
                         + urows_v[bu, pl.ds(16, 16)]
                         * buf[r, pl.ds(qi + 16, 16)]
                         + urows_v[bu, pl.ds(32, 16)]
                         * buf[r, pl.ds(qi + 32, 16)]
                         + urows_v[bu, pl.ds(48, 16)]
                         * buf[r, pl.ds(qi + 48, 16)])
                    acc = jnp.where(lane == k, jnp.sum(t), acc)
                scores_v[pl.ds(c_base + g * LANES, LANES)] = acc

            @pl.when(j + NBUF < NCHUNK)
            def _prefetch():
                pltpu.async_copy(iP.at[iidx_v.at[j + NBUF]], buf, sems[b])
        return carry

    lax.fori_loop(0, NCHUNK // NBUF, ring_body, 0)
    pltpu.sync_copy(scores_v, out.at[pl.ds(wid * CPW, CPW)])


def kernel(user, item, user_table, item_table):
    mesh = plsc.VectorSubcoreMesh(core_axis_name="c", subcore_axis_name="s")
    params = pltpu.CompilerParams(
        needs_layout_passes=False, use_tc_tiling_on_sc=True)

    uP, iP = pl.kernel(
        _tr_body,
        out_type=(jax.ShapeDtypeStruct((NPAIR, W), jnp.float32),
                  jax.ShapeDtypeStruct((NPAIR, W), jnp.float32)),
        mesh=mesh,
        compiler_params=params,
        scratch_types=[
            pltpu.VMEM((4, D, SLAB), jnp.float32),
            pltpu.VMEM((2, SLAB // 2, W), jnp.float32),
            pltpu.VMEM((TAIL_W // 2, W), jnp.float32),
            *([pltpu.SemaphoreType.DMA] * 6),
        ],
    )(user_table.T, item_table.T,
      user_table[TAIL_COL:].reshape(TAIL_W // 2, W),
      item_table[TAIL_COL:].reshape(TAIL_W // 2, W))

    item_f = item.reshape(-1)
    up2d = (user // 2).reshape(B // CHUNK, CHUNK)
    uq2d = ((user % 2) * D).reshape(B // CHUNK, CHUNK)
    ip2d = (item_f // 2).reshape((B * NCAND) // CHUNK, CHUNK)
    iq2d = ((item_f % 2) * D).reshape((B * NCAND) // CHUNK, CHUNK)

    scores = pl.kernel(
        _dot_body,
        out_type=jax.ShapeDtypeStruct((B * NCAND,), jnp.float32),
        mesh=mesh,
        compiler_params=params,
        scratch_types=[
            pltpu.VMEM((UCHUNK, CHUNK), jnp.int32),
            pltpu.VMEM((UCHUNK, CHUNK), jnp.int32),
            pltpu.VMEM((NCHUNK, CHUNK), jnp.int32),
            pltpu.VMEM((NCHUNK, CHUNK), jnp.int32),
            pltpu.VMEM((BPW, D), jnp.float32),
            pltpu.VMEM((NBUF, CHUNK, W), jnp.float32),
            pltpu.VMEM((CPW,), jnp.float32),
            pltpu.SemaphoreType.DMA,
            *([pltpu.SemaphoreType.DMA] * NBUF),
        ],
    )(up2d, uq2d, ip2d, iq2d, uP, iP)
    return scores.reshape(B, NCAND)


# speedup vs baseline: 4.1394x; 2.3927x over previous
import jax
import jax.numpy as jnp
from jax import lax
from jax.experimental import pallas as pl
from jax.experimental.pallas import tpu as pltpu
from jax.experimental.pallas import tpu_sc as plsc

B = 16384
NCAND = 20
D = 64
LANES = 16
NC = 2
NS = 16
NW = NC * NS
BPW = B // NW
CPW = BPW * NCAND
NROWS = 1000000
NPAIR = NROWS // 2
W = 2 * D

SLAB = 256
NFULL = NROWS // SLAB
TAIL_COL = NFULL * SLAB
TAIL_W = NROWS - TAIL_COL
QN = -(-NFULL // NW)
SGROUPS = SLAB // LANES
PITCH = 264
BUFW = D * PITCH

CHUNK = 128
NCHUNK = CPW // CHUNK
UCHUNK = BPW // CHUNK
GROUPS = CHUNK // LANES
NBUF = 2


def _tr_body(utT, itT, tail_u, tail_i, uP, iP, slab_v, sh_v, tail_v, *sems):
    cid = lax.axis_index("c")
    sid = lax.axis_index("s")
    wid = sid * NC + cid
    lane = lax.iota(jnp.int32, LANES)
    half = (lane % 2) * D
    tabs = ((utT, uP), (itT, iP))
    sin = sems[0:4]
    sout = sems[4:6]
    HP = SLAB // 2

    rvs = [(lane + r) % LANES for r in range(LANES)]

    def issue_slab(tab, u, s):
        pltpu.async_copy(tab.at[:, pl.ds(s * SLAB, SLAB)],
                         slab_v.at[u], sin[u])

    def drain_slab(tab, u, s):
        pltpu.make_async_copy(tab.at[:, pl.ds(s * SLAB, SLAB)],
                              slab_v.at[u], sin[u]).wait()

    for u in range(4):
        issue_slab(tabs[u % 2][0], u, wid + (u // 2) * NW)

    def ring(mm, carry):
        for u in range(4):
            tab, outP = tabs[u % 2]
            q = 2 * mm + (u // 2)
            s = wid + q * NW

            @pl.when(s < NFULL)
            def _process():
                drain_slab(tab, u, s)
                @pl.when(s >= wid + NW)
                def _drain_prev():
                    pltpu.make_async_copy(
                        sh_v.at[u % 2],
                        outP.at[pl.ds((s - NW) * HP, HP)],
                        sout[u % 2]).wait()

                sh = sh_v.at[u % 2]

                def group(g, c2):
                    u0 = g * LANES
                    uvec = u0 + lane
                    pvec = uvec // 2
                    for k in range(D // LANES):
                        vs = [plsc.load_gather(slab_v.at[u],
                                               [rvs[r] + (k * LANES), uvec])
                              for r in range(LANES)]
                        for r in range(LANES):
                            plsc.store_scatter(
                                sh, [pvec, half + (rvs[r] + k * LANES)],
                                vs[r])
                    return c2

                lax.fori_loop(0, SGROUPS, group, 0)
                @pl.when(s + 2 * NW < NFULL)
                def _prefetch():
                    issue_slab(tab, u, s + 2 * NW)
                pltpu.async_copy(sh_v.at[u % 2],
                                 outP.at[pl.ds(s * HP, HP)], sout[u % 2])
        return carry

    lax.fori_loop(0, (QN + 1) // 2, ring, 0)

    jl = (NFULL - 1 - wid) // NW
    sl = wid + jl * NW
    for b, (tab, outP) in enumerate(tabs):
        pltpu.make_async_copy(sh_v.at[b],
                              outP.at[pl.ds(sl * HP, HP)],
                              sout[b]).wait()

    @pl.when(wid == 1)
    def _tail():
        for tail_in, outP in ((tail_u, uP), (tail_i, iP)):
            pltpu.sync_copy(tail_in, tail_v)
            pltpu.sync_copy(tail_v,
                            outP.at[pl.ds(TAIL_COL // 2, TAIL_W // 2)])


def _dot_body(up_idx, uq_idx, ip_idx, iq_idx, uP, iP, out,
              uidx_v, uq_v, iidx_v, iq_v, urows_v, irows_v, scores_v,
              sem_u, *sems):
    cid = lax.axis_index("c")
    sid = lax.axis_index("s")
    wid = sid * NC + cid

    pltpu.sync_copy(up_idx.at[pl.ds(wid * UCHUNK, UCHUNK)], uidx_v)
    pltpu.sync_copy(uq_idx.at[pl.ds(wid * UCHUNK, UCHUNK)], uq_v)
    pltpu.sync_copy(ip_idx.at[pl.ds(wid * NCHUNK, NCHUNK)], iidx_v)
    pltpu.sync_copy(iq_idx.at[pl.ds(wid * NCHUNK, NCHUNK)], iq_v)

    for j in range(UCHUNK):
        buf = irows_v.at[j % NBUF]
        pltpu.async_copy(uP.at[uidx_v.at[j]], buf, sem_u).wait()

        def compact(r16, carry):
            qvec = uq_v[j, pl.ds(r16 * 16, 16)]
            for kk in range(16):
                r = r16 * 16 + kk
                qoff = qvec[kk]
                for k in range(D // 16):
                    urows_v[j * CHUNK + r, pl.ds(k * 16, 16)] = (
                        buf[r, pl.ds(qoff + k * 16, 16)])
            return carry

        lax.fori_loop(0, CHUNK // 16, compact, 0)

    for b in range(NBUF):
        pltpu.async_copy(iP.at[iidx_v.at[b]], irows_v.at[b], sems[b])

    lane = lax.iota(jnp.int32, LANES)

    def ring_body(jj, carry):
        for b in range(NBUF):
            j = jj * NBUF + b
            buf = irows_v.at[b]
            pltpu.make_async_copy(iP.at[iidx_v.at[j]], buf, sems[b]).wait()
            c_base = j * CHUNK
            for g in range(GROUPS):
                acc = jnp.zeros((LANES,), jnp.float32)
                qvec = iq_v[j, pl.ds(g * LANES, LANES)]
                for k in range(LANES):
                    r = g * LANES + k
                    bu = (c_base + r) // NCAND
                    qi = qvec[k]
                    t = (urows_v[bu, pl.ds(0, 16)] * buf[r, pl.ds(qi, 16)]
                         + urows_v[bu, pl.ds(16, 16)]
                         * buf[r, pl.ds(qi + 16, 16)]
                         + urows_v[bu, pl.ds(32, 16)]
                         * buf[r, pl.ds(qi + 32, 16)]
                         + urows_v[bu, pl.ds(48, 16)]
                         * buf[r, pl.ds(qi + 48, 16)])
                    acc = jnp.where(lane == k, jnp.sum(t), acc)
                scores_v[pl.ds(c_base + g * LANES, LANES)] = acc

            @pl.when(j + NBUF < NCHUNK)
            def _prefetch():
                pltpu.async_copy(iP.at[iidx_v.at[j + NBUF]], buf, sems[b])
        return carry

    lax.fori_loop(0, NCHUNK // NBUF, ring_body, 0)
    pltpu.sync_copy(scores_v, out.at[pl.ds(wid * CPW, CPW)])


def kernel(user, item, user_table, item_table):
    mesh = plsc.VectorSubcoreMesh(core_axis_name="c", subcore_axis_name="s")
    params = pltpu.CompilerParams(
        needs_layout_passes=False, use_tc_tiling_on_sc=True)

    uP, iP = pl.kernel(
        _tr_body,
        out_type=(jax.ShapeDtypeStruct((NPAIR, W), jnp.float32),
                  jax.ShapeDtypeStruct((NPAIR, W), jnp.float32)),
        mesh=mesh,
        compiler_params=params,
        scratch_types=[
            pltpu.VMEM((4, D, SLAB), jnp.float32),
            pltpu.VMEM((2, SLAB // 2, W), jnp.float32),
            pltpu.VMEM((TAIL_W // 2, W), jnp.float32),
            *([pltpu.SemaphoreType.DMA] * 6),
        ],
    )(user_table.T, item_table.T,
      user_table[TAIL_COL:].reshape(TAIL_W // 2, W),
      item_table[TAIL_COL:].reshape(TAIL_W // 2, W))

    item_f = item.reshape(-1)
    up2d = (user // 2).reshape(B // CHUNK, CHUNK)
    uq2d = ((user % 2) * D).reshape(B // CHUNK, CHUNK)
    ip2d = (item_f // 2).reshape((B * NCAND) // CHUNK, CHUNK)
    iq2d = ((item_f % 2) * D).reshape((B * NCAND) // CHUNK, CHUNK)

    scores = pl.kernel(
        _dot_body,
        out_type=jax.ShapeDtypeStruct((B * NCAND,), jnp.float32),
        mesh=mesh,
        compiler_params=params,
        scratch_types=[
            pltpu.VMEM((UCHUNK, CHUNK), jnp.int32),
            pltpu.VMEM((UCHUNK, CHUNK), jnp.int32),
            pltpu.VMEM((NCHUNK, CHUNK), jnp.int32),
            pltpu.VMEM((NCHUNK, CHUNK), jnp.int32),
            pltpu.VMEM((BPW, D), jnp.float32),
            pltpu.VMEM((NBUF, CHUNK, W), jnp.float32),
            pltpu.VMEM((CPW,), jnp.float32),
            pltpu.SemaphoreType.DMA,
            *([pltpu.SemaphoreType.DMA] * NBUF),
        ],
    )(up2d, uq2d, ip2d, iq2d, uP, iP)
    return scores.reshape(B, NCAND)


# speedup vs baseline: 4.4484x; 1.0746x over previous
import jax
import jax.numpy as jnp
from jax import lax
from jax.experimental import pallas as pl
from jax.experimental.pallas import tpu as pltpu
from jax.experimental.pallas import tpu_sc as plsc

B = 16384
NCAND = 20
D = 64
LANES = 16
NC = 2
NS = 16
NW = NC * NS
BPW = B // NW
CPW = BPW * NCAND
NROWS = 1000000
NPAIR = NROWS // 2
W = 2 * D

SLAB = 256
NFULL = NROWS // SLAB
TAIL_COL = NFULL * SLAB
TAIL_W = NROWS - TAIL_COL
QN = -(-NFULL // NW)
SGROUPS = SLAB // LANES
PITCH = 264
BUFW = D * PITCH

CHUNK = 128
NCHUNK = CPW // CHUNK
UCHUNK = BPW // CHUNK
GROUPS = CHUNK // LANES
NBUF = 2


def _tr_body(utT, itT, tail_u, tail_i, uP, iP, slab_v, sh_v, tail_v, *sems):
    cid = lax.axis_index("c")
    sid = lax.axis_index("s")
    wid = sid * NC + cid
    lane = lax.iota(jnp.int32, LANES)
    half = (lane % 2) * D
    tabs = ((utT, uP), (itT, iP))
    sin = sems[0:4]
    sout = sems[4:6]
    HP = SLAB // 2

    rvs = [(lane + r) % LANES for r in range(LANES)]

    def issue_slab(tab, u, s):
        pltpu.async_copy(tab.at[:, pl.ds(s * SLAB, SLAB)],
                         slab_v.at[u], sin[u])

    def drain_slab(tab, u, s):
        pltpu.make_async_copy(tab.at[:, pl.ds(s * SLAB, SLAB)],
                              slab_v.at[u], sin[u]).wait()

    for u in range(4):
        issue_slab(tabs[u % 2][0], u, wid + (u // 2) * NW)

    def ring(mm, carry):
        for u in range(4):
            tab, outP = tabs[u % 2]
            q = 2 * mm + (u // 2)
            s = wid + q * NW

            @pl.when(s < NFULL)
            def _process():
                drain_slab(tab, u, s)
                @pl.when(s >= wid + NW)
                def _drain_prev():
                    pltpu.make_async_copy(
                        sh_v.at[u % 2],
                        outP.at[pl.ds((s - NW) * HP, HP)],
                        sout[u % 2]).wait()

                sh = sh_v.at[u % 2]

                def group(g, c2):
                    u0 = g * LANES
                    uvec = u0 + lane
                    pvec = uvec // 2
                    for k in range(D // LANES):
                        vs = [plsc.load_gather(slab_v.at[u],
                                               [rvs[r] + (k * LANES), uvec])
                              for r in range(LANES)]
                        for r in range(LANES):
                            plsc.store_scatter(
                                sh, [pvec, half + (rvs[r] + k * LANES)],
                                vs[r])
                    return c2

                lax.fori_loop(0, SGROUPS, group, 0)
                @pl.when(s + 2 * NW < NFULL)
                def _prefetch():
                    issue_slab(tab, u, s + 2 * NW)
                pltpu.async_copy(sh_v.at[u % 2],
                                 outP.at[pl.ds(s * HP, HP)], sout[u % 2])
        return carry

    lax.fori_loop(0, (QN + 1) // 2, ring, 0)

    jl = (NFULL - 1 - wid) // NW
    sl = wid + jl * NW
    for b, (tab, outP) in enumerate(tabs):
        pltpu.make_async_copy(sh_v.at[b],
                              outP.at[pl.ds(sl * HP, HP)],
                              sout[b]).wait()

    @pl.when(wid == 1)
    def _tail():
        for tail_in, outP in ((tail_u, uP), (tail_i, iP)):
            pltpu.sync_copy(tail_in, tail_v)
            pltpu.sync_copy(tail_v,
                            outP.at[pl.ds(TAIL_COL // 2, TAIL_W // 2)])


def _dot_body(u_idx, i_idx, uR, iR, out,
              uidx_v, iidx_v, urows_v, irows_v, scores_v, sem_u, *sems):
    cid = lax.axis_index("c")
    sid = lax.axis_index("s")
    wid = sid * NC + cid

    pltpu.sync_copy(u_idx.at[pl.ds(wid * UCHUNK, UCHUNK)], uidx_v)
    pltpu.sync_copy(i_idx.at[pl.ds(wid * NCHUNK, NCHUNK)], iidx_v)

    for j in range(UCHUNK):
        pltpu.async_copy(uR.at[uidx_v.at[j]],
                         urows_v.at[pl.ds(j * CHUNK, CHUNK)], sem_u)
    for j in range(UCHUNK):
        pltpu.make_async_copy(uR.at[uidx_v.at[j]],
                              urows_v.at[pl.ds(j * CHUNK, CHUNK)],
                              sem_u).wait()

    for b in range(NBUF):
        pltpu.async_copy(iR.at[iidx_v.at[b]], irows_v.at[b], sems[b])

    lane = lax.iota(jnp.int32, LANES)

    def ring_body(jj, carry):
        for b in range(NBUF):
            j = jj * NBUF + b
            buf = irows_v.at[b]
            pltpu.make_async_copy(iR.at[iidx_v.at[j]], buf, sems[b]).wait()
            c_base = j * CHUNK
            for g in range(GROUPS):
                acc = jnp.zeros((LANES,), jnp.float32)
                for k in range(LANES):
                    r = g * LANES + k
                    bu = (c_base + r) // NCAND
                    t = (urows_v[bu, pl.ds(0, 16)] * buf[r, pl.ds(0, 16)]
                         + urows_v[bu, pl.ds(16, 16)] * buf[r, pl.ds(16, 16)]
                         + urows_v[bu, pl.ds(32, 16)] * buf[r, pl.ds(32, 16)]
                         + urows_v[bu, pl.ds(48, 16)] * buf[r, pl.ds(48, 16)])
                    acc = jnp.where(lane == k, jnp.sum(t), acc)
                scores_v[pl.ds(c_base + g * LANES, LANES)] = acc

            @pl.when(j + NBUF < NCHUNK)
            def _prefetch():
                pltpu.async_copy(iR.at[iidx_v.at[j + NBUF]], buf, sems[b])
        return carry

    lax.fori_loop(0, NCHUNK // NBUF, ring_body, 0)
    pltpu.sync_copy(scores_v, out.at[pl.ds(wid * CPW, CPW)])


def kernel(user, item, user_table, item_table):
    mesh = plsc.VectorSubcoreMesh(core_axis_name="c", subcore_axis_name="s")
    params = pltpu.CompilerParams(
        needs_layout_passes=False, use_tc_tiling_on_sc=True)

    uP, iP = pl.kernel(
        _tr_body,
        out_type=(jax.ShapeDtypeStruct((NPAIR, W), jnp.float32),
                  jax.ShapeDtypeStruct((NPAIR, W), jnp.float32)),
        mesh=mesh,
        compiler_params=params,
        scratch_types=[
            pltpu.VMEM((4, D, SLAB), jnp.float32),
            pltpu.VMEM((2, SLAB // 2, W), jnp.float32),
            pltpu.VMEM((TAIL_W // 2, W), jnp.float32),
            *([pltpu.SemaphoreType.DMA] * 6),
        ],
    )(user_table.T, item_table.T,
      user_table[TAIL_COL:].reshape(TAIL_W // 2, W),
      item_table[TAIL_COL:].reshape(TAIL_W // 2, W))

    uR = uP.reshape(NROWS, D)
    iR = iP.reshape(NROWS, D)
    user2d = user.reshape(B // CHUNK, CHUNK)
    item2d = item.reshape((B * NCAND) // CHUNK, CHUNK)

    scores = pl.kernel(
        _dot_body,
        out_type=jax.ShapeDtypeStruct((B * NCAND,), jnp.float32),
        mesh=mesh,
        compiler_params=pltpu.CompilerParams(
            needs_layout_passes=False, use_tc_tiling_on_sc=False),
        scratch_types=[
            pltpu.VMEM((UCHUNK, CHUNK), jnp.int32),
            pltpu.VMEM((NCHUNK, CHUNK), jnp.int32),
            pltpu.VMEM((BPW, D), jnp.float32),
            pltpu.VMEM((NBUF, CHUNK, D), jnp.float32),
            pltpu.VMEM((CPW,), jnp.float32),
            pltpu.SemaphoreType.DMA,
            *([pltpu.SemaphoreType.DMA] * NBUF),
        ],
    )(user2d, item2d, uR, iR)
    return scores.reshape(B, NCAND)
